# baseline (speedup 1.0000x reference)
import jax
import jax.numpy as jnp
from jax import lax
from jax.experimental import pallas as pl
from jax.experimental.pallas import tpu as pltpu
from jax.experimental.pallas import tpu_sc as plsc

N = 100000
C = 4
NUM_WORKERS = 32
LANES = 16
NODES_PER_W = N // NUM_WORKERS
CHUNK = 3328
ITERS = CHUNK // LANES
N_MAIN = (N // 128) * 128
TAIL = N - N_MAIN
MAX_BASE = N_MAIN - CHUNK

NB = 4
SUBS = (256, 1024, 1024, 1024)
OFFS = (0, 256, 1280, 2304)
I_LO = 8
I_HI3 = 48
UNROLL = 4

_LN2 = 0.6931471805599453
_C0 = 2.2159764891e-07
_C1 = 9.9997024330e-01
_C2 = -4.9933394898e-01
_C3 = 3.2751171370e-01
_C4 = -2.2396689943e-01
_C5 = 1.3198966240e-01
_C6 = -5.3267477734e-02
_C7 = 1.0243828631e-02


def _log_f32(s):
    bits = lax.bitcast_convert_type(s, jnp.int32)
    e = (bits >> 23) - 127
    mbits = (bits & 0x7FFFFF) | 0x3F800000
    m = lax.bitcast_convert_type(mbits, jnp.float32)
    z = m - 1.0
    z2 = z * z
    z4 = z2 * z2
    ln_m = ((_C0 + _C1 * z) + (_C2 + _C3 * z) * z2
            + ((_C4 + _C5 * z) + (_C6 + _C7 * z) * z2) * z4)
    return e.astype(jnp.float32) * _LN2 + ln_m


def _node_losses(pv, mv, lv, sl):
    p0 = pv[0, sl]
    p1 = pv[1, sl]
    p2 = pv[2, sl]
    p3 = pv[3, sl]
    m0 = mv[0, sl]
    m1 = mv[1, sl]
    m2 = mv[2, sl]
    m3 = mv[3, sl]
    lbl = lv[sl]
    mx = jnp.maximum(jnp.maximum(p0, p1), jnp.maximum(p2, p3))
    s = ((jnp.exp(p0 - mx) + jnp.exp(p1 - mx))
         + (jnp.exp(p2 - mx) + jnp.exp(p3 - mx)))
    lse = _log_f32(s) + mx
    p_lbl = jnp.where(lbl == 0, p0,
                      jnp.where(lbl == 1, p1,
                                jnp.where(lbl == 2, p2, p3)))
    ce = lse - p_lbl
    mdot = (m0 * p0 + m1 * p1) + (m2 * p2 + m3 * p3)
    unl = lse - mdot
    return ce, unl, lbl


def _sc_body(pred_hbm, lbl_hbm, marg_hbm, out_hbm,
             pred0_v, marg0_v, lbl0_v, pred1_v, marg1_v, lbl1_v,
             pred2_v, marg2_v, lbl2_v, pred3_v, marg3_v, lbl3_v,
             tp_v, tm_v, tl_v, row_v, sem0, sem1, sem2, sem3, semt):
    pred_b = (pred0_v, pred1_v, pred2_v, pred3_v)
    marg_b = (marg0_v, marg1_v, marg2_v, marg3_v)
    lbl_b = (lbl0_v, lbl1_v, lbl2_v, lbl3_v)
    sems = (sem0, sem1, sem2, sem3)
    wid = lax.axis_index("s") * 2 + lax.axis_index("c")
    own_lo = wid * NODES_PER_W
    own_hi = jnp.minimum(own_lo + NODES_PER_W, N_MAIN)
    base = pl.multiple_of(
        jnp.minimum((own_lo // 128) * 128, MAX_BASE), 128)

    def quarter_copies(h):
        bh = pl.multiple_of(base + OFFS[h], 128)
        return [
            pltpu.make_async_copy(pred_hbm.at[:, pl.ds(bh, SUBS[h])],
                                  pred_b[h], sems[h]),
            pltpu.make_async_copy(marg_hbm.at[:, pl.ds(bh, SUBS[h])],
                                  marg_b[h], sems[h]),
            pltpu.make_async_copy(lbl_hbm.at[pl.ds(bh, SUBS[h])],
                                  lbl_b[h], sems[h]),
        ]

    tail_copies = [
        pltpu.make_async_copy(pred_hbm.at[:, pl.ds(N_MAIN, TAIL)], tp_v, semt),
        pltpu.make_async_copy(marg_hbm.at[:, pl.ds(N_MAIN, TAIL)], tm_v, semt),
        pltpu.make_async_copy(lbl_hbm.at[pl.ds(N_MAIN, TAIL)], tl_v, semt),
    ]
    copies = [quarter_copies(h) for h in range(NB)]

    for cp in copies[0]:
        cp.start()
    for cp in tail_copies:
        cp.start()

    lane = lax.iota(jnp.int32, LANES)
    zero = jnp.zeros((LANES,), jnp.float32)

    def make_masked(h):
        def masked_body(i, carry):
            ps, pc, us, uc = carry
            ce, unl, lbl = _node_losses(pred_b[h], marg_b[h],
                                        lbl_b[h], pl.ds(i * LANES, LANES))
            g = base + OFFS[h] + i * LANES + lane
            valid = (g >= own_lo) & (g < own_hi)
            posf = jnp.where(valid & (lbl > 0), 1.0, 0.0)
            unlf = jnp.where(valid & (lbl == 0), 1.0, 0.0)
            return (ps + ce * posf, pc + posf, us + unl * unlf, uc + unlf)
        return masked_body

    def run_interior(h, lo, hi, carry):
        @plsc.parallel_loop(lo, hi, unroll=UNROLL, carry=carry)
        def interior(i, c):
            ps, pc, us, uc = c
            ce, unl, lbl = _node_losses(pred_b[h], marg_b[h],
                                        lbl_b[h], pl.ds(i * LANES, LANES))
            posf = jnp.where(lbl > 0, 1.0, 0.0)
            return (ps + ce * posf, pc + posf,
                    us + (unl - unl * posf), uc + (1.0 - posf))
        return interior

    acc = (zero, zero, zero, zero)
    for h in range(NB):
        with jax.named_scope(f"dma_wait{h}"):
            for cp in copies[h]:
                cp.wait()
        if h + 1 < NB:
            for cp in copies[h + 1]:
                cp.start()
        sub_iters = SUBS[h] // LANES
        lo = I_LO if h == 0 else 0
        hi = I_HI3 if h == NB - 1 else sub_iters
        if lo > 0:
            with jax.named_scope(f"masked_lo{h}"):
                acc = lax.fori_loop(0, lo, make_masked(h), acc)
        with jax.named_scope(f"interior{h}"):
            acc = run_interior(h, lo, hi, acc)
        if hi < sub_iters:
            with jax.named_scope(f"masked_hi{h}"):
                acc = lax.fori_loop(hi, sub_iters, make_masked(h), acc)

    def tail_body(j, carry):
        ps, pc, us, uc = carry
        ce, unl, lbl = _node_losses(tp_v, tm_v, tl_v,
                                    pl.ds(j * LANES, LANES))
        mine = wid == (NUM_WORKERS - 1)
        posf = jnp.where(mine & (lbl > 0), 1.0, 0.0)
        unlf = jnp.where(mine & (lbl == 0), 1.0, 0.0)
        return (ps + ce * posf, pc + posf, us + unl * unlf, uc + unlf)

    with jax.named_scope("tail"):
        for cp in tail_copies:
            cp.wait()
        acc = lax.fori_loop(0, TAIL // LANES, tail_body, acc)

    ps, pc, us, uc = acc
    pss = jnp.sum(ps, axis=0)
    pcs = jnp.sum(pc, axis=0)
    uss = jnp.sum(us, axis=0)
    ucs = jnp.sum(uc, axis=0)

    packed = (jnp.where(lane == 0, pss, 0.0)
              + jnp.where(lane == 1, pcs, 0.0)
              + jnp.where(lane == 2, uss, 0.0)
              + jnp.where(lane == 3, ucs, 0.0))
    row_v[...] = packed
    pltpu.sync_copy(row_v, out_hbm.at[wid])


@jax.jit
def _hope_loss(pred_t, labels_i32, marg_t):
    mesh = plsc.VectorSubcoreMesh(core_axis_name="c", subcore_axis_name="s")
    partials = pl.kernel(
        _sc_body,
        out_type=jax.ShapeDtypeStruct((NUM_WORKERS, LANES), jnp.float32),
        mesh=mesh,
        scratch_types=(
            [v for sub in SUBS for v in
             (pltpu.VMEM((C, sub), jnp.float32),
              pltpu.VMEM((C, sub), jnp.float32),
              pltpu.VMEM((sub,), jnp.int32))]
            + [pltpu.VMEM((C, TAIL), jnp.float32),
               pltpu.VMEM((C, TAIL), jnp.float32),
               pltpu.VMEM((TAIL,), jnp.int32),
               pltpu.VMEM((LANES,), jnp.float32)]
            + [pltpu.SemaphoreType.DMA] * (NB + 1)
        ),
        compiler_params=pltpu.CompilerParams(
            needs_layout_passes=False,
            disable_bounds_checks=True,
            disable_semaphore_checks=True,
        ),
    )(pred_t, labels_i32, marg_t)
    tot = partials[0]
    for i in range(1, NUM_WORKERS):
        tot = tot + partials[i]
    pos_loss = tot[0] / jnp.maximum(tot[1], 1.0)
    unl_loss = tot[2] / jnp.maximum(tot[3], 1.0)
    return pos_loss + unl_loss


def kernel(predictions, labels, marginals):
    return _hope_loss(
        predictions.T,
        labels.astype(jnp.int32),
        marginals.T.astype(jnp.float32),
    )


# speedup vs baseline: 1.0277x; 1.0277x over previous
import jax
import jax.numpy as jnp
from jax import lax
from jax.experimental import pallas as pl
from jax.experimental.pallas import tpu as pltpu
from jax.experimental.pallas import tpu_sc as plsc

N = 100000
C = 4
NUM_WORKERS = 32
LANES = 16
NODES_PER_W = N // NUM_WORKERS
CHUNK = 3328
ITERS = CHUNK // LANES
N_MAIN = (N // 128) * 128
TAIL = N - N_MAIN
MAX_BASE = N_MAIN - CHUNK

NB = 4
SUBS = (896, 896, 896, 640)
OFFS = (0, 896, 1792, 2688)
I_LO = 8
I_HI3 = 24
UNROLL = 2

_LN2 = 0.6931471805599453
_C0 = 2.2159764891e-07
_C1 = 9.9997024330e-01
_C2 = -4.9933394898e-01
_C3 = 3.2751171370e-01
_C4 = -2.2396689943e-01
_C5 = 1.3198966240e-01
_C6 = -5.3267477734e-02
_C7 = 1.0243828631e-02


def _log_f32(s):
    bits = lax.bitcast_convert_type(s, jnp.int32)
    e = (bits >> 23) - 127
    mbits = (bits & 0x7FFFFF) | 0x3F800000
    m = lax.bitcast_convert_type(mbits, jnp.float32)
    z = m - 1.0
    z2 = z * z
    z4 = z2 * z2
    ln_m = ((_C0 + _C1 * z) + (_C2 + _C3 * z) * z2
            + ((_C4 + _C5 * z) + (_C6 + _C7 * z) * z2) * z4)
    return e.astype(jnp.float32) * _LN2 + ln_m


def _node_losses(pv, mv, lv, sl):
    p0 = pv[0, sl]
    p1 = pv[1, sl]
    p2 = pv[2, sl]
    p3 = pv[3, sl]
    m0 = mv[0, sl]
    m1 = mv[1, sl]
    m2 = mv[2, sl]
    m3 = mv[3, sl]
    lbl = lv[sl]
    mx = jnp.maximum(jnp.maximum(p0, p1), jnp.maximum(p2, p3))
    s = ((jnp.exp(p0 - mx) + jnp.exp(p1 - mx))
         + (jnp.exp(p2 - mx) + jnp.exp(p3 - mx)))
    lse = _log_f32(s) + mx
    p_lbl = jnp.where(lbl == 0, p0,
                      jnp.where(lbl == 1, p1,
                                jnp.where(lbl == 2, p2, p3)))
    ce = lse - p_lbl
    mdot = (m0 * p0 + m1 * p1) + (m2 * p2 + m3 * p3)
    unl = lse - mdot
    return ce, unl, lbl


def _sc_body(pred_hbm, lbl_hbm, marg_hbm, out_hbm,
             pred0_v, marg0_v, lbl0_v, pred1_v, marg1_v, lbl1_v,
             pred2_v, marg2_v, lbl2_v, pred3_v, marg3_v, lbl3_v,
             tp_v, tm_v, tl_v, row_v, sem0, sem1, sem2, sem3, semt):
    pred_b = (pred0_v, pred1_v, pred2_v, pred3_v)
    marg_b = (marg0_v, marg1_v, marg2_v, marg3_v)
    lbl_b = (lbl0_v, lbl1_v, lbl2_v, lbl3_v)
    sems = (sem0, sem1, sem2, sem3)
    wid = lax.axis_index("s") * 2 + lax.axis_index("c")
    own_lo = wid * NODES_PER_W
    own_hi = jnp.minimum(own_lo + NODES_PER_W, N_MAIN)
    base = pl.multiple_of(
        jnp.minimum((own_lo // 128) * 128, MAX_BASE), 128)

    def quarter_copies(h):
        bh = pl.multiple_of(base + OFFS[h], 128)
        return [
            pltpu.make_async_copy(pred_hbm.at[:, pl.ds(bh, SUBS[h])],
                                  pred_b[h], sems[h]),
            pltpu.make_async_copy(marg_hbm.at[:, pl.ds(bh, SUBS[h])],
                                  marg_b[h], sems[h]),
            pltpu.make_async_copy(lbl_hbm.at[pl.ds(bh, SUBS[h])],
                                  lbl_b[h], sems[h]),
        ]

    tail_copies = [
        pltpu.make_async_copy(pred_hbm.at[:, pl.ds(N_MAIN, TAIL)], tp_v, semt),
        pltpu.make_async_copy(marg_hbm.at[:, pl.ds(N_MAIN, TAIL)], tm_v, semt),
        pltpu.make_async_copy(lbl_hbm.at[pl.ds(N_MAIN, TAIL)], tl_v, semt),
    ]
    copies = [quarter_copies(h) for h in range(NB)]

    for cp in copies[0]:
        cp.start()
    for cp in tail_copies:
        cp.start()

    lane = lax.iota(jnp.int32, LANES)
    zero = jnp.zeros((LANES,), jnp.float32)

    def make_masked(h):
        def masked_body(i, carry):
            ps, pc, us, uc = carry
            ce, unl, lbl = _node_losses(pred_b[h], marg_b[h],
                                        lbl_b[h], pl.ds(i * LANES, LANES))
            g = base + OFFS[h] + i * LANES + lane
            valid = (g >= own_lo) & (g < own_hi)
            posf = jnp.where(valid & (lbl > 0), 1.0, 0.0)
            unlf = jnp.where(valid & (lbl == 0), 1.0, 0.0)
            return (ps + ce * posf, pc + posf, us + unl * unlf, uc + unlf)
        return masked_body

    def run_interior(h, lo, hi, carry):
        @plsc.parallel_loop(lo, hi, unroll=UNROLL, carry=carry)
        def interior(i, c):
            ps, pc, us, uc = c
            ce, unl, lbl = _node_losses(pred_b[h], marg_b[h],
                                        lbl_b[h], pl.ds(i * LANES, LANES))
            posf = jnp.where(lbl > 0, 1.0, 0.0)
            return (ps + ce * posf, pc + posf,
                    us + (unl - unl * posf), uc + (1.0 - posf))
        return interior

    acc = (zero, zero, zero, zero)
    for h in range(NB):
        for cp in copies[h]:
            cp.wait()
        if h + 1 < NB:
            for cp in copies[h + 1]:
                cp.start()
        sub_iters = SUBS[h] // LANES
        lo = I_LO if h == 0 else 0
        hi = I_HI3 if h == NB - 1 else sub_iters
        if lo > 0:
            acc = lax.fori_loop(0, lo, make_masked(h), acc)
        acc = run_interior(h, lo, hi, acc)
        if hi < sub_iters:
            acc = lax.fori_loop(hi, sub_iters, make_masked(h), acc)

    def tail_body(j, carry):
        ps, pc, us, uc = carry
        ce, unl, lbl = _node_losses(tp_v, tm_v, tl_v,
                                    pl.ds(j * LANES, LANES))
        mine = wid == (NUM_WORKERS - 1)
        posf = jnp.where(mine & (lbl > 0), 1.0, 0.0)
        unlf = jnp.where(mine & (lbl == 0), 1.0, 0.0)
        return (ps + ce * posf, pc + posf, us + unl * unlf, uc + unlf)

    for cp in tail_copies:
        cp.wait()
    acc = lax.fori_loop(0, TAIL // LANES, tail_body, acc)

    ps, pc, us, uc = acc
    pss = jnp.sum(ps, axis=0)
    pcs = jnp.sum(pc, axis=0)
    uss = jnp.sum(us, axis=0)
    ucs = jnp.sum(uc, axis=0)

    packed = (jnp.where(lane == 0, pss, 0.0)
              + jnp.where(lane == 1, pcs, 0.0)
              + jnp.where(lane == 2, uss, 0.0)
              + jnp.where(lane == 3, ucs, 0.0))
    row_v[...] = packed
    pltpu.sync_copy(row_v, out_hbm.at[wid])


@jax.jit
def _hope_loss(pred_t, labels_i32, marg_t):
    mesh = plsc.VectorSubcoreMesh(core_axis_name="c", subcore_axis_name="s")
    partials = pl.kernel(
        _sc_body,
        out_type=jax.ShapeDtypeStruct((NUM_WORKERS, LANES), jnp.float32),
        mesh=mesh,
        scratch_types=(
            [v for sub in SUBS for v in
             (pltpu.VMEM((C, sub), jnp.float32),
              pltpu.VMEM((C, sub), jnp.float32),
              pltpu.VMEM((sub,), jnp.int32))]
            + [pltpu.VMEM((C, TAIL), jnp.float32),
               pltpu.VMEM((C, TAIL), jnp.float32),
               pltpu.VMEM((TAIL,), jnp.int32),
               pltpu.VMEM((LANES,), jnp.float32)]
            + [pltpu.SemaphoreType.DMA] * (NB + 1)
        ),
        compiler_params=pltpu.CompilerParams(
            needs_layout_passes=False,
            disable_bounds_checks=True,
            disable_semaphore_checks=True,
        ),
    )(pred_t, labels_i32, marg_t)
    tot = partials[0]
    for i in range(1, NUM_WORKERS):
        tot = tot + partials[i]
    pos_loss = tot[0] / jnp.maximum(tot[1], 1.0)
    unl_loss = tot[2] / jnp.maximum(tot[3], 1.0)
    return pos_loss + unl_loss


def kernel(predictions, labels, marginals):
    return _hope_loss(
        predictions.T,
        labels.astype(jnp.int32),
        marginals.T.astype(jnp.float32),
    )
